# baseline (speedup 1.0000x reference)
import functools

import jax
import jax.numpy as jnp
from jax import lax
from jax.experimental import pallas as pl
from jax.experimental.pallas import tpu as pltpu
from jax.experimental.pallas import tpu_sc as plsc

_D = 32
_NB = 16384
_NT = 50
_NC = 2
_NS = 16
_NW = _NC * _NS
_BW = _NB // _NW
_V = 1000000
_NTILE = 7812
_CPW = _NTILE // _NW + 1
_TAIL = _V - 128


@functools.partial(
    pl.kernel,
    mesh=plsc.VectorSubcoreMesh(core_axis_name="c", subcore_axis_name="s"),
    out_type=jax.ShapeDtypeStruct((_V * _D,), jnp.float32),
    scratch_types=[
        pltpu.VMEM((2, _D, 128), jnp.float32),
        pltpu.VMEM((2, 128 * _D), jnp.float32),
        pltpu.VMEM((_D, 64), jnp.float32),
        pltpu.VMEM((64 * _D,), jnp.float32),
        pltpu.SemaphoreType.DMA,
        pltpu.SemaphoreType.DMA,
        pltpu.SemaphoreType.DMA,
        pltpu.SemaphoreType.DMA,
    ],
    compiler_params=pltpu.CompilerParams(
        use_tc_tiling_on_sc=True, needs_layout_passes=False
    ),
)
def _canon_kernel(emb_t_hbm, out_hbm, slab_v, rows_v, slab64_v, rows64_v,
                  sl0, sl1, sw0, sw1):
    wid = lax.axis_index("s") * _NC + lax.axis_index("c")
    sl = (sl0, sl1)
    sw = (sw0, sw1)

    def chunk_info(k):
        c = wid + k * _NW
        cb = pl.multiple_of(c * 128, 128)
        return c < _NTILE, cb

    def start_load(k, s):
        act, cb = chunk_info(k)

        @pl.when(act)
        def _():
            pltpu.async_copy(
                emb_t_hbm.at[pl.ds(0, _D), pl.ds(cb, 128)], slab_v.at[s], sl[s]
            )

    start_load(0, 0)

    def outer(kk, carry):
        for half in range(2):
            k = kk * 2 + half
            s = half
            act, cb = chunk_info(k)
            start_load(k + 1, 1 - s)

            @pl.when(act)
            def _():
                pltpu.make_async_copy(
                    emb_t_hbm.at[pl.ds(0, _D), pl.ds(cb, 128)], slab_v.at[s], sl[s]
                ).wait()

                @pl.when(k >= 2)
                def _():
                    pltpu.make_async_copy(
                        rows_v.at[s], out_hbm.at[pl.ds(cb * _D, 128 * _D)], sw[s]
                    ).wait()

                def rloop(ro, c2):
                    for rr in range(8):
                        r = ro * 8 + rr
                        for h2 in range(2):
                            idx_d = lax.iota(jnp.int32, 16) + 16 * h2
                            idx_c = jnp.full((16,), r, jnp.int32)
                            vec = plsc.load_gather(slab_v.at[s], [idx_d, idx_c])
                            rows_v[s, pl.ds((2 * r + h2) * 16, 16)] = vec
                    return c2

                lax.fori_loop(0, 16, rloop, 0)
                pltpu.async_copy(
                    rows_v.at[s], out_hbm.at[pl.ds(cb * _D, 128 * _D)], sw[s]
                )
        return carry

    lax.fori_loop(0, (_CPW + 1) // 2, outer, 0)

    @pl.when(wid == 0)
    def _tail():
        pltpu.sync_copy(
            emb_t_hbm.at[pl.ds(0, _D), pl.ds(_NTILE * 128, 64)], slab64_v
        )

        def rloop64(ro, c2):
            for rr in range(8):
                r = ro * 8 + rr
                for h2 in range(2):
                    idx_d = lax.iota(jnp.int32, 16) + 16 * h2
                    idx_c = jnp.full((16,), r, jnp.int32)
                    vec = plsc.load_gather(slab64_v, [idx_d, idx_c])
                    rows64_v[pl.ds((2 * r + h2) * 16, 16)] = vec
            return c2

        lax.fori_loop(0, 8, rloop64, 0)
        pltpu.sync_copy(rows64_v, out_hbm.at[pl.ds(_NTILE * 128 * _D, 64 * _D)])

    for s in range(2):
        pltpu.make_async_copy(
            rows_v.at[s], out_hbm.at[pl.ds(0, 128 * _D)], sw[s]
        ).wait()


@functools.partial(
    pl.kernel,
    mesh=plsc.VectorSubcoreMesh(core_axis_name="c", subcore_axis_name="s"),
    out_type=jax.ShapeDtypeStruct((_NT, _NB, _D), jnp.float32),
    scratch_types=[
        pltpu.VMEM((2, _BW), jnp.int32),
        pltpu.VMEM((2, _BW, _D), jnp.float32),
        pltpu.SemaphoreType.DMA,
        pltpu.SemaphoreType.DMA,
        pltpu.SemaphoreType.DMA,
        pltpu.SemaphoreType.DMA,
        pltpu.SemaphoreType.DMA,
    ],
    compiler_params=pltpu.CompilerParams(use_tc_tiling_on_sc=False),
)
def _gather_kernel(ids_hbm, table_hbm, out_hbm, idx_v, rows_v, si0, si1, sg, ss0, ss1):
    wid = lax.axis_index("s") * _NC + lax.axis_index("c")
    b0 = wid * _BW
    si = (si0, si1)
    ss = (ss0, ss1)

    pltpu.async_copy(ids_hbm.at[0, pl.ds(b0, _BW)], idx_v.at[0], si0)

    def outer(tt, carry):
        for k in range(2):
            t = tt * 2 + k

            @pl.when(t < _NT - 1)
            def _prefetch():
                pltpu.async_copy(
                    ids_hbm.at[t + 1, pl.ds(b0, _BW)], idx_v.at[1 - k], si[1 - k]
                )

            pltpu.make_async_copy(
                ids_hbm.at[t, pl.ds(b0, _BW)], idx_v.at[k], si[k]
            ).wait()

            @pl.when(t >= 2)
            def _drain():
                pltpu.make_async_copy(
                    rows_v.at[k], out_hbm.at[t, pl.ds(b0, _BW)], ss[k]
                ).wait()

            pltpu.async_copy(table_hbm.at[idx_v.at[k]], rows_v.at[k], sg).wait()

            pltpu.async_copy(rows_v.at[k], out_hbm.at[t, pl.ds(b0, _BW)], ss[k])
        return carry

    lax.fori_loop(0, _NT // 2, outer, 0)

    pltpu.make_async_copy(rows_v.at[0], out_hbm.at[_NT - 2, pl.ds(b0, _BW)], ss0).wait()
    pltpu.make_async_copy(rows_v.at[1], out_hbm.at[_NT - 1, pl.ds(b0, _BW)], ss1).wait()


def kernel(token_ids, embedding):
    ids_t = jnp.transpose(token_ids).astype(jnp.int32)
    table_lin = _canon_kernel(jnp.transpose(embedding))
    out_t = _gather_kernel(ids_t, table_lin.reshape(_V, _D))
    return jnp.transpose(out_t, (1, 0, 2))


# speedup vs baseline: 1.3176x; 1.3176x over previous
import functools

import jax
import jax.numpy as jnp
from jax import lax
from jax.experimental import pallas as pl
from jax.experimental.pallas import tpu as pltpu
from jax.experimental.pallas import tpu_sc as plsc

_D = 32
_NB = 16384
_NT = 50
_NC = 2
_NS = 16
_NW = _NC * _NS
_BW = _NB // _NW


@functools.partial(
    pl.kernel,
    mesh=plsc.VectorSubcoreMesh(core_axis_name="c", subcore_axis_name="s"),
    out_type=jax.ShapeDtypeStruct((_NT, _NB, _D), jnp.float32),
    scratch_types=[
        pltpu.VMEM((2, _BW), jnp.int32),
        pltpu.VMEM((2, _BW, _D), jnp.float32),
        pltpu.SemaphoreType.DMA,
        pltpu.SemaphoreType.DMA,
        pltpu.SemaphoreType.DMA,
        pltpu.SemaphoreType.DMA,
        pltpu.SemaphoreType.DMA,
    ],
    compiler_params=pltpu.CompilerParams(use_tc_tiling_on_sc=False),
)
def _gather_kernel(ids_hbm, table_hbm, out_hbm, idx_v, rows_v, si0, si1, sg, ss0, ss1):
    wid = lax.axis_index("s") * _NC + lax.axis_index("c")
    b0 = wid * _BW
    si = (si0, si1)
    ss = (ss0, ss1)

    pltpu.async_copy(ids_hbm.at[0, pl.ds(b0, _BW)], idx_v.at[0], si0)

    def outer(tt, carry):
        for k in range(2):
            t = tt * 2 + k

            @pl.when(t < _NT - 1)
            def _prefetch():
                pltpu.async_copy(
                    ids_hbm.at[t + 1, pl.ds(b0, _BW)], idx_v.at[1 - k], si[1 - k]
                )

            pltpu.make_async_copy(
                ids_hbm.at[t, pl.ds(b0, _BW)], idx_v.at[k], si[k]
            ).wait()

            @pl.when(t >= 2)
            def _drain():
                pltpu.make_async_copy(
                    rows_v.at[k], out_hbm.at[t, pl.ds(b0, _BW)], ss[k]
                ).wait()

            pltpu.async_copy(table_hbm.at[idx_v.at[k]], rows_v.at[k], sg).wait()

            pltpu.async_copy(rows_v.at[k], out_hbm.at[t, pl.ds(b0, _BW)], ss[k])
        return carry

    lax.fori_loop(0, _NT // 2, outer, 0)

    pltpu.make_async_copy(rows_v.at[0], out_hbm.at[_NT - 2, pl.ds(b0, _BW)], ss0).wait()
    pltpu.make_async_copy(rows_v.at[1], out_hbm.at[_NT - 1, pl.ds(b0, _BW)], ss1).wait()


def kernel(token_ids, embedding):
    ids_t = jnp.transpose(token_ids).astype(jnp.int32)
    out_t = _gather_kernel(ids_t, embedding)
    return jnp.transpose(out_t, (1, 0, 2))
